# baseline (speedup 1.0000x reference)
import functools

import jax
import jax.numpy as jnp
from jax import lax
from jax.experimental import pallas as pl
from jax.experimental import layout as jax_layout
from jax.experimental.pallas import tpu as pltpu
from jax.experimental.pallas import tpu_sc as plsc

_B = 4096
_D = 64
_C = 5024
_V = 1000000

_BN = 512
_NB = _B // _BN

_LINEAR2D = jax_layout.Layout(major_to_minor=(1, 0), tiling=((8,),))


_BATCH = 8


@functools.lru_cache(maxsize=1)
def _make_sc_gather():
    info = plsc.get_sparse_core_info()
    nc, ns = info.num_cores, info.num_subcores
    nw = nc * ns
    bpw = _B // nw
    nk = bpw // 16
    mesh = plsc.VectorSubcoreMesh(core_axis_name="c", subcore_axis_name="s")

    @functools.partial(
        pl.kernel,
        mesh=mesh,
        compiler_params=pltpu.CompilerParams(needs_layout_passes=False),
        out_type=[
            jax.ShapeDtypeStruct((nw, _D, bpw), jnp.float32),
            jax.ShapeDtypeStruct((nw, _D, bpw), jnp.float32),
        ],
        scratch_types=[
            pltpu.VMEM((bpw,), jnp.int32),
            pltpu.VMEM((bpw,), jnp.int32),
            pltpu.VMEM((_BATCH, _D, 128), jnp.float32),
            pltpu.VMEM((_D, bpw), jnp.float32),
            pltpu.VMEM((_D, bpw), jnp.float32),
            pltpu.VMEM((_D, bpw), jnp.float32),
            pltpu.SemaphoreType.DMA,
        ],
    )
    def sc_gather(uid_hbm, iid_hbm, ut_hbm, itc_hbm, itn_hbm,
                  ui_out, nt_out,
                  uidv, iidv, slab, ubuf, ibuf, nbuf, sem):
        wid = lax.axis_index("s") * nc + lax.axis_index("c")
        pltpu.sync_copy(uid_hbm.at[wid], uidv)
        pltpu.sync_copy(iid_hbm.at[wid], iidv)
        rows = [lax.iota(jnp.int32, 16) + 16 * k for k in range(4)]

        def gather_table(tbl_hbm, idv, outbuf):
            def batch_body(bi, _):
                chunk = idv[pl.ds(bi * 16, 16)]
                for half in range(16 // _BATCH):
                    waves = []
                    for jj in range(_BATCH):
                        v = chunk[half * _BATCH + jj]
                        base = pl.multiple_of((v >> 7) * 128, 128)
                        c = pltpu.make_async_copy(
                            tbl_hbm.at[:, pl.ds(base, 128)], slab.at[jj], sem)
                        c.start()
                        waves.append(c)
                    for c in waves:
                        c.wait()
                    for jj in range(_BATCH):
                        j = bi * 16 + half * _BATCH + jj
                        v = chunk[half * _BATCH + jj]
                        col = jnp.full((16,), 1, jnp.int32) * (v & 127)
                        jcol = jnp.full((16,), 1, jnp.int32) * j
                        for k in range(4):
                            vals = plsc.load_gather(slab.at[jj],
                                                    [rows[k], col])
                            plsc.store_scatter(outbuf, [rows[k], jcol], vals)
                return 0

            lax.fori_loop(0, bpw // 16, batch_body, 0)

        gather_table(itn_hbm, iidv, nbuf)
        pltpu.sync_copy(nbuf, nt_out.at[wid])
        gather_table(ut_hbm, uidv, ubuf)
        gather_table(itc_hbm, iidv, ibuf)

        def prod(d, _):
            for k in range(nk):
                s = pl.ds(k * 16, 16)
                ubuf[d, s] = ubuf[d, s] * ibuf[d, s]
            return 0

        lax.fori_loop(0, _D, prod, 0)
        pltpu.sync_copy(ubuf, ui_out.at[wid])

    return sc_gather


def _matmul_body(wc_ref, cft_ref, out_ref):
    out_ref[...] = lax.dot_general(
        wc_ref[...], cft_ref[...],
        (((1,), (0,)), ((), ())),
        preferred_element_type=jnp.float32)


def _matmul(content_fc_w, cft):
    return pl.pallas_call(
        _matmul_body,
        grid=(_NB,),
        in_specs=[
            pl.BlockSpec((_D, _C), lambda j: (0, 0)),
            pl.BlockSpec((_C, _BN), lambda j: (0, j)),
        ],
        out_specs=pl.BlockSpec((_D, _BN), lambda j: (0, j)),
        out_shape=jax.ShapeDtypeStruct((_D, _B), jnp.float32),
        compiler_params=pltpu.CompilerParams(
            dimension_semantics=("arbitrary",)),
    )(content_fc_w, cft)


def _combine_body(matt_ref, nt_ref, ui_ref, cb_ref, cw_ref, hybw_ref,
                  cbias_ref, hbias_ref, out_ref):
    hw0 = hybw_ref[0, 0]
    hw1 = hybw_ref[0, 1]
    content_pred = jnp.sum(nt_ref[0] * (matt_ref[...] + cb_ref[...]),
                           axis=0, keepdims=True)
    collab_pred = jnp.sum(ui_ref[0] * cw_ref[...], axis=0,
                          keepdims=True) + cbias_ref[0, 0]
    out_ref[0] = hw0 * collab_pred + hw1 * content_pred + hbias_ref[0, 0]


def _combine(matt, nt3, ui3, content_fc_b, collab_fc_w, hybrid_fc_w,
             collab_fc_b, hybrid_fc_b, nw, bpw):
    full = lambda shape: pl.BlockSpec(shape, lambda w: tuple(0 for _ in shape))
    return pl.pallas_call(
        _combine_body,
        grid=(nw,),
        in_specs=[
            pl.BlockSpec((_D, bpw), lambda w: (0, w)),
            pl.BlockSpec((1, _D, bpw), lambda w: (w, 0, 0)),
            pl.BlockSpec((1, _D, bpw), lambda w: (w, 0, 0)),
            full((_D, 1)),
            full((_D, 1)),
            full((1, 2)),
            full((1, 1)),
            full((1, 1)),
        ],
        out_specs=pl.BlockSpec((1, 1, bpw), lambda w: (w, 0, 0)),
        out_shape=jax.ShapeDtypeStruct((nw, 1, bpw), jnp.float32),
        compiler_params=pltpu.CompilerParams(
            dimension_semantics=("arbitrary",)),
    )(matt, nt3, ui3, content_fc_b.reshape(_D, 1), collab_fc_w.reshape(_D, 1),
      hybrid_fc_w, collab_fc_b.reshape(1, 1), hybrid_fc_b.reshape(1, 1))


def kernel(user_id, item_id, content_features, user_table, item_table_collab,
           collab_fc_w, collab_fc_b, item_table_content, content_fc_w,
           content_fc_b, hybrid_fc_w, hybrid_fc_b):
    info = plsc.get_sparse_core_info()
    nw = info.num_cores * info.num_subcores
    bpw = _B // nw
    uid = user_id.astype(jnp.int32)
    iid = item_id.astype(jnp.int32)
    utt = user_table.T
    itct = item_table_collab.T
    itnt = item_table_content.T
    cft = content_features.T

    matt = _matmul(content_fc_w, cft)
    ui3, nt3 = _make_sc_gather()(
        uid.reshape(nw, bpw), iid.reshape(nw, bpw), utt, itct, itnt)
    out = _combine(matt, nt3, ui3, content_fc_b, collab_fc_w, hybrid_fc_w,
                   collab_fc_b, hybrid_fc_b, nw, bpw)
    return out.reshape(_B)


# speedup vs baseline: 1.0525x; 1.0525x over previous
import functools

import jax
import jax.numpy as jnp
from jax import lax
from jax.experimental import pallas as pl
from jax.experimental import layout as jax_layout
from jax.experimental.pallas import tpu as pltpu
from jax.experimental.pallas import tpu_sc as plsc

_B = 4096
_D = 64
_C = 5024
_V = 1000000

_BN = 1024
_NB = _B // _BN
_CW = 4

_LINEAR2D = jax_layout.Layout(major_to_minor=(1, 0), tiling=((8,),))


_BATCH = 8


@functools.lru_cache(maxsize=1)
def _make_sc_gather():
    info = plsc.get_sparse_core_info()
    nc, ns = info.num_cores, info.num_subcores
    nw = nc * ns
    bpw = _B // nw
    nk = bpw // 16
    mesh = plsc.VectorSubcoreMesh(core_axis_name="c", subcore_axis_name="s")

    @functools.partial(
        pl.kernel,
        mesh=mesh,
        compiler_params=pltpu.CompilerParams(needs_layout_passes=False),
        out_type=[
            jax.ShapeDtypeStruct((nw, _D, bpw), jnp.float32),
            jax.ShapeDtypeStruct((nw, _D, bpw), jnp.float32),
        ],
        scratch_types=[
            pltpu.VMEM((bpw,), jnp.int32),
            pltpu.VMEM((bpw,), jnp.int32),
            pltpu.VMEM((_BATCH, _D, 128), jnp.float32),
            pltpu.VMEM((_D, bpw), jnp.float32),
            pltpu.VMEM((_D, bpw), jnp.float32),
            pltpu.VMEM((_D, bpw), jnp.float32),
            pltpu.SemaphoreType.DMA,
        ],
    )
    def sc_gather(uid_hbm, iid_hbm, ut_hbm, itc_hbm, itn_hbm,
                  ui_out, nt_out,
                  uidv, iidv, slab, ubuf, ibuf, nbuf, sem):
        wid = lax.axis_index("s") * nc + lax.axis_index("c")
        pltpu.sync_copy(uid_hbm.at[wid], uidv)
        pltpu.sync_copy(iid_hbm.at[wid], iidv)
        rows = [lax.iota(jnp.int32, 16) + 16 * k for k in range(4)]

        def gather_table(tbl_hbm, idv, outbuf):
            def batch_body(bi, _):
                chunk = idv[pl.ds(bi * 16, 16)]
                for half in range(16 // _BATCH):
                    waves = []
                    for jj in range(_BATCH):
                        v = chunk[half * _BATCH + jj]
                        base = pl.multiple_of((v >> 7) * 128, 128)
                        c = pltpu.make_async_copy(
                            tbl_hbm.at[:, pl.ds(base, 128)], slab.at[jj], sem)
                        c.start()
                        waves.append(c)
                    for c in waves:
                        c.wait()
                    for jj in range(_BATCH):
                        j = bi * 16 + half * _BATCH + jj
                        v = chunk[half * _BATCH + jj]
                        col = jnp.full((16,), 1, jnp.int32) * (v & 127)
                        jcol = jnp.full((16,), 1, jnp.int32) * j
                        for k in range(4):
                            vals = plsc.load_gather(slab.at[jj],
                                                    [rows[k], col])
                            plsc.store_scatter(outbuf, [rows[k], jcol], vals)
                return 0

            lax.fori_loop(0, bpw // 16, batch_body, 0)

        gather_table(itn_hbm, iidv, nbuf)
        pltpu.sync_copy(nbuf, nt_out.at[wid])
        gather_table(ut_hbm, uidv, ubuf)
        gather_table(itc_hbm, iidv, ibuf)

        def prod(d, _):
            for k in range(nk):
                s = pl.ds(k * 16, 16)
                ubuf[d, s] = ubuf[d, s] * ibuf[d, s]
            return 0

        lax.fori_loop(0, _D, prod, 0)
        pltpu.sync_copy(ubuf, ui_out.at[wid])

    return sc_gather


def _matmul_body(wc_ref, cft_ref, out_ref):
    out_ref[...] = lax.dot_general(
        wc_ref[...], cft_ref[...],
        (((1,), (0,)), ((), ())),
        preferred_element_type=jnp.float32)


def _matmul(content_fc_w, cft):
    return pl.pallas_call(
        _matmul_body,
        grid=(_NB,),
        in_specs=[
            pl.BlockSpec((_D, _C), lambda j: (0, 0)),
            pl.BlockSpec((_C, _BN), lambda j: (0, j)),
        ],
        out_specs=pl.BlockSpec((_D, _BN), lambda j: (0, j)),
        out_shape=jax.ShapeDtypeStruct((_D, _B), jnp.float32),
        compiler_params=pltpu.CompilerParams(
            dimension_semantics=("arbitrary",)),
    )(content_fc_w, cft)


def _combine_body(matt_ref, nt_ref, ui_ref, cb_ref, cw_ref, hybw_ref,
                  cbias_ref, hbias_ref, out_ref):
    hw0 = hybw_ref[0, 0]
    hw1 = hybw_ref[0, 1]
    for t in range(_CW):
        bpw = nt_ref.shape[2]
        matt = matt_ref[:, pl.ds(t * bpw, bpw)]
        content_pred = jnp.sum(nt_ref[t] * (matt + cb_ref[...]),
                               axis=0, keepdims=True)
        collab_pred = jnp.sum(ui_ref[t] * cw_ref[...], axis=0,
                              keepdims=True) + cbias_ref[0, 0]
        out_ref[t] = hw0 * collab_pred + hw1 * content_pred + hbias_ref[0, 0]


def _combine(matt, nt3, ui3, content_fc_b, collab_fc_w, hybrid_fc_w,
             collab_fc_b, hybrid_fc_b, nw, bpw):
    full = lambda shape: pl.BlockSpec(shape, lambda w: tuple(0 for _ in shape))
    return pl.pallas_call(
        _combine_body,
        grid=(nw // _CW,),
        in_specs=[
            pl.BlockSpec((_D, _CW * bpw), lambda w: (0, w)),
            pl.BlockSpec((_CW, _D, bpw), lambda w: (w, 0, 0)),
            pl.BlockSpec((_CW, _D, bpw), lambda w: (w, 0, 0)),
            full((_D, 1)),
            full((_D, 1)),
            full((1, 2)),
            full((1, 1)),
            full((1, 1)),
        ],
        out_specs=pl.BlockSpec((_CW, 1, bpw), lambda w: (w, 0, 0)),
        out_shape=jax.ShapeDtypeStruct((nw, 1, bpw), jnp.float32),
        compiler_params=pltpu.CompilerParams(
            dimension_semantics=("arbitrary",)),
    )(matt, nt3, ui3, content_fc_b.reshape(_D, 1), collab_fc_w.reshape(_D, 1),
      hybrid_fc_w, collab_fc_b.reshape(1, 1), hybrid_fc_b.reshape(1, 1))


def kernel(user_id, item_id, content_features, user_table, item_table_collab,
           collab_fc_w, collab_fc_b, item_table_content, content_fc_w,
           content_fc_b, hybrid_fc_w, hybrid_fc_b):
    info = plsc.get_sparse_core_info()
    nw = info.num_cores * info.num_subcores
    bpw = _B // nw
    uid = user_id.astype(jnp.int32)
    iid = item_id.astype(jnp.int32)
    utt = user_table.T
    itct = item_table_collab.T
    itnt = item_table_content.T
    cft = content_features.T

    matt = _matmul(content_fc_w, cft)
    ui3, nt3 = _make_sc_gather()(
        uid.reshape(nw, bpw), iid.reshape(nw, bpw), utt, itct, itnt)
    out = _combine(matt, nt3, ui3, content_fc_b, collab_fc_w, hybrid_fc_w,
                   collab_fc_b, hybrid_fc_b, nw, bpw)
    return out.reshape(_B)
